# baseline (speedup 1.0000x reference)
import functools

import jax
import jax.numpy as jnp
from jax import lax
from jax.experimental import pallas as pl
from jax.experimental.pallas import tpu as pltpu
from jax.experimental.pallas import tpu_sc as plsc

BATCH = 2
SEQ = 4096
D_MODEL = 2048
NC = 2
NS = 16
NW = NC * NS
B_PER_W = BATCH * SEQ // NW
W_PER_BATCH = SEQ // B_PER_W
CHUNK = 16
NCHUNK = B_PER_W // CHUNK
NBUF = 3
LA = NBUF - 1

_mesh = plsc.VectorSubcoreMesh(core_axis_name="c", subcore_axis_name="s")


@functools.partial(
    pl.kernel,
    out_type=jax.ShapeDtypeStruct((BATCH, SEQ, D_MODEL), jnp.float32),
    mesh=_mesh,
    scratch_types=[
        pltpu.VMEM((B_PER_W,), jnp.int32),
        pltpu.VMEM((NBUF, CHUNK, D_MODEL), jnp.float32),
        pltpu.SemaphoreType.DMA((NBUF,)),
        pltpu.SemaphoreType.DMA((NBUF,)),
    ],
)
def _sc_gather(idx_hbm, table_hbm, out_hbm, idx_v, bufs, gsem, ssem):
    wid = lax.axis_index("s") * NC + lax.axis_index("c")
    b = wid // W_PER_BATCH
    seq0 = (wid % W_PER_BATCH) * B_PER_W
    pltpu.sync_copy(idx_hbm.at[b, pl.ds(seq0, B_PER_W)], idx_v)
    gath = [None] * NCHUNK
    outc = [None] * NCHUNK
    for t in range(NCHUNK + LA):
        if t < NCHUNK:
            buf = t % NBUF
            if t >= NBUF:
                outc[t - NBUF].wait()
            gath[t] = pltpu.async_copy(
                table_hbm.at[idx_v.at[pl.ds(t * CHUNK, CHUNK)]],
                bufs.at[buf], gsem.at[buf])
        j = t - LA
        if j >= 0:
            gath[j].wait()
            outc[j] = pltpu.async_copy(
                bufs.at[j % NBUF],
                out_hbm.at[b, pl.ds(seq0 + j * CHUNK, CHUNK)],
                ssem.at[j % NBUF])
    for j in range(NCHUNK - NBUF, NCHUNK):
        outc[j].wait()


def kernel(input_ids, embed_table):
    return _sc_gather(input_ids, embed_table)


# speedup vs baseline: 1.0093x; 1.0093x over previous
import functools

import jax
import jax.numpy as jnp
from jax import lax
from jax.experimental import pallas as pl
from jax.experimental.pallas import tpu as pltpu
from jax.experimental.pallas import tpu_sc as plsc

BATCH = 2
SEQ = 4096
D_MODEL = 2048
NC = 2
NS = 16
NW = NC * NS
B_PER_W = BATCH * SEQ // NW
W_PER_BATCH = SEQ // B_PER_W
CHUNK = 8
NCHUNK = B_PER_W // CHUNK
NBUF = 6
LA = NBUF - 1

_mesh = plsc.VectorSubcoreMesh(core_axis_name="c", subcore_axis_name="s")


@functools.partial(
    pl.kernel,
    out_type=jax.ShapeDtypeStruct((BATCH, SEQ, D_MODEL), jnp.float32),
    mesh=_mesh,
    scratch_types=[
        pltpu.VMEM((B_PER_W,), jnp.int32),
        pltpu.VMEM((NBUF, CHUNK, D_MODEL), jnp.float32),
        pltpu.SemaphoreType.DMA((NBUF,)),
        pltpu.SemaphoreType.DMA((NBUF,)),
    ],
)
def _sc_gather(idx_hbm, table_hbm, out_hbm, idx_v, bufs, gsem, ssem):
    wid = lax.axis_index("s") * NC + lax.axis_index("c")
    b = wid // W_PER_BATCH
    seq0 = (wid % W_PER_BATCH) * B_PER_W
    pltpu.sync_copy(idx_hbm.at[b, pl.ds(seq0, B_PER_W)], idx_v)
    gath = [None] * NCHUNK
    outc = [None] * NCHUNK
    for t in range(NCHUNK + LA):
        if t < NCHUNK:
            buf = t % NBUF
            if t >= NBUF:
                outc[t - NBUF].wait()
            gath[t] = pltpu.async_copy(
                table_hbm.at[idx_v.at[pl.ds(t * CHUNK, CHUNK)]],
                bufs.at[buf], gsem.at[buf])
        j = t - LA
        if j >= 0:
            gath[j].wait()
            outc[j] = pltpu.async_copy(
                bufs.at[j % NBUF],
                out_hbm.at[b, pl.ds(seq0 + j * CHUNK, CHUNK)],
                ssem.at[j % NBUF])
    for j in range(NCHUNK - NBUF, NCHUNK):
        outc[j].wait()


def kernel(input_ids, embed_table):
    return _sc_gather(input_ids, embed_table)
